# Initial kernel scaffold; baseline (speedup 1.0000x reference)
import jax
import jax.numpy as jnp
from jax.experimental import pallas as pl


def kernel(features, positions, W1, b1, W2, b2, lambda_weight, temperature, spatial_weight):
    raise NotImplementedError("write your pallas kernel here")


# speedup vs baseline: 7.7900x; 7.7900x over previous
import functools

import jax
import jax.numpy as jnp
from jax import lax
from jax.experimental import pallas as pl
from jax.experimental.pallas import tpu as pltpu
from jax.experimental.pallas import tpu_sc as plsc

N = 4096
D = 512
HID = 256
MAXC = 50
KNN = 20
E = N * KNN
SPATIAL_DECAY = 50.0
THRESH = 0.1
KC = 50
CP = 128
SP = 64
RB = 256
RB2 = 512
NSC = 2
NTILE = 16
NW = NSC * NTILE
RPT = N // NW
EPW = E // NW
CH = 128
CAP = E
PB = 128
NPB = E // (PB * 128)


def _topk_body(scal_ref, fb_ref, fa_ref, pb_ref, pt_ref, w1_ref,
               val_ref, idx_ref, row_ref, deg_ref, h1_ref, lam_ref):
    pid = pl.program_id(0)
    lam = jax.nn.sigmoid(scal_ref[0, 0])
    temp = scal_ref[0, 1]

    fb = fb_ref[...]
    fa = fa_ref[...]
    sb = 1.0 / jnp.maximum(jnp.sqrt(jnp.sum(fb * fb, axis=1, keepdims=True)), 1e-12)
    sa = 1.0 / jnp.maximum(jnp.sqrt(jnp.sum(fa * fa, axis=1, keepdims=True)), 1e-12)
    sim = lax.dot_general(fb * sb, fa * sa, (((1,), (1,)), ((), ())),
                          preferred_element_type=jnp.float32)

    xb = pb_ref[:, 0:1]
    yb = pb_ref[:, 1:2]
    xa = pt_ref[0:1, :]
    ya = pt_ref[1:2, :]
    d2 = (xb - xa) ** 2 + (yb - ya) ** 2
    spatial = jnp.exp(d2 * (-1.0 / (2.0 * SPATIAL_DECAY ** 2)))

    adj = jax.nn.sigmoid((lam * sim + (1.0 - lam) * spatial) * temp)

    cols = lax.broadcasted_iota(jnp.int32, (RB, N), 1)
    a = adj
    vals = []
    idxs = []
    for _ in range(KNN):
        m = jnp.max(a, axis=1, keepdims=True)
        i = jnp.min(jnp.where(a == m, cols, N), axis=1, keepdims=True)
        vals.append(m)
        idxs.append(i)
        a = jnp.where(cols == i, -jnp.inf, a)
    tv = jnp.concatenate(vals, axis=1)
    ti = jnp.concatenate(idxs, axis=1)

    bsel = (a == -jnp.inf).astype(jnp.float32)
    dcol = jnp.sum(bsel, axis=0, keepdims=True)

    iw = ti
    sidx = []
    sval = []
    for _ in range(KNN):
        mi = jnp.min(iw, axis=1, keepdims=True)
        hit = iw == mi
        sval.append(jnp.sum(jnp.where(hit, tv, 0.0), axis=1, keepdims=True))
        sidx.append(mi)
        iw = jnp.where(hit, jnp.int32(N), iw)
    val_ref[...] = jnp.concatenate(sval, axis=1)
    idx_ref[...] = jnp.concatenate(sidx, axis=1)
    row_ref[...] = lax.broadcasted_iota(jnp.int32, (RB, KNN), 0) + pid * RB

    h1_ref[...] = jnp.dot(fb, w1_ref[...], preferred_element_type=jnp.float32)

    @pl.when(pid == 0)
    def _():
        deg_ref[...] = jnp.zeros((1, N), jnp.float32)
        lam_ref[...] = jnp.full((1, 1), 0.0) + lam
    deg_ref[...] += dcol


def _run_topk(features, positions, pos_t, W1, scal):
    grid = N // RB
    return pl.pallas_call(
        _topk_body,
        grid=(grid,),
        in_specs=[
            pl.BlockSpec((1, 2), lambda i: (0, 0)),
            pl.BlockSpec((RB, D), lambda i: (i, 0)),
            pl.BlockSpec((N, D), lambda i: (0, 0)),
            pl.BlockSpec((RB, 2), lambda i: (i, 0)),
            pl.BlockSpec((2, N), lambda i: (0, 0)),
            pl.BlockSpec((D, HID), lambda i: (0, 0)),
        ],
        out_specs=[
            pl.BlockSpec((RB, KNN), lambda i: (i, 0)),
            pl.BlockSpec((RB, KNN), lambda i: (i, 0)),
            pl.BlockSpec((RB, KNN), lambda i: (i, 0)),
            pl.BlockSpec((1, N), lambda i: (0, 0)),
            pl.BlockSpec((RB, HID), lambda i: (i, 0)),
            pl.BlockSpec((1, 1), lambda i: (0, 0)),
        ],
        out_shape=[
            jax.ShapeDtypeStruct((N, KNN), jnp.float32),
            jax.ShapeDtypeStruct((N, KNN), jnp.int32),
            jax.ShapeDtypeStruct((N, KNN), jnp.int32),
            jax.ShapeDtypeStruct((1, N), jnp.float32),
            jax.ShapeDtypeStruct((N, HID), jnp.float32),
            jax.ShapeDtypeStruct((1, 1), jnp.float32),
        ],
    )(scal, features, features, positions, pos_t, W1)


def _pos_body(dst_ref, pos_ref, pack_ref, counts_ref, sent_ref, carry):
    pid = pl.program_id(0)

    @pl.when(pid == 0)
    def _():
        carry[...] = jnp.zeros((1, NW), jnp.float32)

    dstb = dst_ref[...]
    owner = jnp.right_shift(dstb, 7)
    ri = lax.broadcasted_iota(jnp.int32, (PB, PB), 0)
    ci = lax.broadcasted_iota(jnp.int32, (PB, PB), 1)
    uinc = (ri <= ci).astype(jnp.float32)
    lstr = (ri > ci).astype(jnp.float32)

    posf = jnp.zeros((PB, 128), jnp.float32)
    tots = []
    for o in range(NW):
        eq = (owner == o).astype(jnp.float32)
        pref = jnp.dot(eq, uinc, preferred_element_type=jnp.float32)
        rowtot = pref[:, 127:128]
        rowoffs = jnp.dot(lstr, rowtot, preferred_element_type=jnp.float32)
        base = carry[0:1, o:o + 1]
        posf += eq * (pref - 1.0 + rowoffs + base + float(o) * float(CAP))
        tots.append(rowoffs[PB - 1:PB, 0:1] + rowtot[PB - 1:PB, 0:1])
    carry[...] += jnp.concatenate(tots, axis=1)
    pos_ref[...] = posf.astype(jnp.int32)

    cnt_col = lax.dot_general(_eye(NW), carry[...], (((1,), (1,)), ((), ())),
                              preferred_element_type=jnp.float32)
    sent_ref[...] = (cnt_col.astype(jnp.int32)
                     + lax.broadcasted_iota(jnp.int32, (NW, CH), 1)
                     + lax.broadcasted_iota(jnp.int32, (NW, CH), 0) * CAP)

    eids = (lax.broadcasted_iota(jnp.int32, (PB, 128), 0) * 128
            + lax.broadcasted_iota(jnp.int32, (PB, 128), 1) + pid * (PB * 128))
    src = eids // KNN
    dl = dstb & 127
    pack_ref[...] = src * 256 + dl
    counts_ref[...] = carry[...].astype(jnp.int32)


def _run_pos(dst2d):
    return pl.pallas_call(
        _pos_body,
        grid=(NPB,),
        in_specs=[pl.BlockSpec((PB, 128), lambda i: (i, 0))],
        out_specs=[
            pl.BlockSpec((PB, 128), lambda i: (i, 0)),
            pl.BlockSpec((PB, 128), lambda i: (i, 0)),
            pl.BlockSpec((1, NW), lambda i: (0, 0)),
            pl.BlockSpec((NW, CH), lambda i: (0, 0)),
        ],
        out_shape=[
            jax.ShapeDtypeStruct((E // 128, 128), jnp.int32),
            jax.ShapeDtypeStruct((E // 128, 128), jnp.int32),
            jax.ShapeDtypeStruct((1, NW), jnp.int32),
            jax.ShapeDtypeStruct((NW, CH), jnp.int32),
        ],
        scratch_shapes=[pltpu.VMEM((1, NW), jnp.float32)],
    )(dst2d)


def _sc_bucket(pos, pack, sentpos):
    mesh = plsc.VectorSubcoreMesh(core_axis_name="c", subcore_axis_name="s")

    @functools.partial(
        pl.kernel,
        out_type=jax.ShapeDtypeStruct((NW * CAP + CH,), jnp.int32),
        mesh=mesh,
        scratch_types=[
            pltpu.VMEM((CH,), jnp.int32),
            pltpu.VMEM((CH,), jnp.int32),
            pltpu.VMEM((CH,), jnp.int32),
            pltpu.VMEM((CH,), jnp.int32),
            pltpu.SemaphoreType.DMA,
        ],
    )
    def k(pos_hbm, pack_hbm, sent_hbm, gl_hbm, posv, packv, sentv, sposv, sem):
        cid = lax.axis_index("c")
        sid = lax.axis_index("s")
        w = cid * NTILE + sid

        def s1(ci, _):
            base = w * EPW + ci * CH
            pltpu.sync_copy(pos_hbm.at[pl.ds(base, CH)], posv)
            pltpu.sync_copy(pack_hbm.at[pl.ds(base, CH)], packv)
            pltpu.async_copy(packv, gl_hbm.at[posv], sem).wait()
            return 0
        lax.fori_loop(0, EPW // CH, s1, 0)

        def fill(v, _):
            sentv[pl.ds(v * 16, 16)] = jnp.full((16,), RPT, jnp.int32)
            return 0
        lax.fori_loop(0, CH // 16, fill, 0)
        pltpu.sync_copy(sent_hbm.at[w], sposv)
        pltpu.async_copy(sentv, gl_hbm.at[sposv], sem).wait()

    return k(pos, pack, sentpos)


def _sc_aggregate(table, glpack, counts16, dp):
    mesh = plsc.VectorSubcoreMesh(core_axis_name="c", subcore_axis_name="s")

    @functools.partial(
        pl.kernel,
        out_type=jax.ShapeDtypeStruct((N, dp), jnp.float32),
        mesh=mesh,
        scratch_types=[
            pltpu.VMEM((CH,), jnp.int32),
            pltpu.VMEM((CH,), jnp.int32),
            pltpu.VMEM((CH,), jnp.int32),
            pltpu.VMEM((16,), jnp.int32),
            pltpu.VMEM((CH, dp), jnp.float32),
            pltpu.VMEM((RPT + 16, dp), jnp.float32),
            pltpu.SemaphoreType.DMA,
        ],
    )
    def k(table_hbm, gl_hbm, cnt_hbm, out_hbm,
          packv, srcslab, dlslab, cntv, rows, acc, sem):
        cid = lax.axis_index("c")
        sid = lax.axis_index("s")
        w = cid * NTILE + sid

        def z1(i, _):
            def z2(j, _):
                acc[i, pl.ds(j * 16, 16)] = jnp.zeros((16,), jnp.float32)
                return 0
            return lax.fori_loop(0, dp // 16, z2, 0)
        lax.fori_loop(0, RPT + 16, z1, 0)

        pltpu.sync_copy(cnt_hbm.at[w], cntv)
        cnt = cntv[...][0]
        nslab = (cnt + CH - 1) // CH
        sh8 = jnp.full((16,), 8, jnp.int32)
        m255 = jnp.full((16,), 255, jnp.int32)

        def slab(sl, _):
            pltpu.sync_copy(gl_hbm.at[pl.ds(w * CAP + sl * CH, CH)], packv)

            def unp(v, _):
                pk = packv[pl.ds(v * 16, 16)]
                srcslab[pl.ds(v * 16, 16)] = lax.shift_right_logical(pk, sh8)
                dlslab[pl.ds(v * 16, 16)] = pk & m255
                return 0
            lax.fori_loop(0, CH // 16, unp, 0)
            pltpu.async_copy(table_hbm.at[srcslab], rows, sem).wait()

            def grp(g, _):
                dv = dlslab[pl.ds(g * 16, 16)]
                for l in range(16):
                    dr = dv[l]

                    def colblk(j, _, _dr=dr, _r=g * 16 + l):
                        plsc.addupdate(acc.at[_dr, pl.ds(j * 16, 16)],
                                       rows[_r, pl.ds(j * 16, 16)])
                        return 0
                    lax.fori_loop(0, dp // 16, colblk, 0)
                return 0
            lax.fori_loop(0, CH // 16, grp, 0)
            return 0
        lax.fori_loop(0, nslab, slab, 0)

        pltpu.sync_copy(acc.at[pl.ds(0, RPT)], out_hbm.at[pl.ds(w * RPT, RPT)])

    return k(table, glpack, counts16)


def _eye(n):
    return (lax.broadcasted_iota(jnp.int32, (n, n), 0)
            == lax.broadcasted_iota(jnp.int32, (n, n), 1)).astype(jnp.float32)


def _prep_body(deg_ref, h1_ref, dinv_ref, h1d_ref):
    degr = deg_ref[0:1, :] + 1.0
    dinvr = lax.rsqrt(jnp.maximum(degr, 1.0))
    dinv_c = lax.dot_general(_eye(RB2), dinvr, (((1,), (1,)), ((), ())),
                             preferred_element_type=jnp.float32)
    dinv_ref[...] = dinv_c
    h1d_ref[...] = h1_ref[...] * dinv_c


def _run_prep(deg, h1):
    grid = N // RB2
    return pl.pallas_call(
        _prep_body,
        grid=(grid,),
        in_specs=[
            pl.BlockSpec((1, RB2), lambda i: (0, i)),
            pl.BlockSpec((RB2, HID), lambda i: (i, 0)),
        ],
        out_specs=[
            pl.BlockSpec((RB2, 1), lambda i: (i, 0)),
            pl.BlockSpec((RB2, HID), lambda i: (i, 0)),
        ],
        out_shape=[
            jax.ShapeDtypeStruct((N, 1), jnp.float32),
            jax.ShapeDtypeStruct((N, HID), jnp.float32),
        ],
    )(deg, h1)


def _layer2_body(g1_ref, h1d_ref, dinv_ref, b1_ref, w2_ref, h2d_ref):
    dinv = dinv_ref[...]
    g = g1_ref[...] + h1d_ref[...]
    h = jnp.maximum(dinv * g + b1_ref[...], 0.0)
    h2d_ref[...] = dinv * jnp.dot(h, w2_ref[...], preferred_element_type=jnp.float32)


def _run_layer2(g1, h1d, dinv, b1r, w2p):
    grid = N // RB2
    return pl.pallas_call(
        _layer2_body,
        grid=(grid,),
        in_specs=[
            pl.BlockSpec((RB2, HID), lambda i: (i, 0)),
            pl.BlockSpec((RB2, HID), lambda i: (i, 0)),
            pl.BlockSpec((RB2, 1), lambda i: (i, 0)),
            pl.BlockSpec((1, HID), lambda i: (0, 0)),
            pl.BlockSpec((HID, CP), lambda i: (0, 0)),
        ],
        out_specs=pl.BlockSpec((RB2, CP), lambda i: (i, 0)),
        out_shape=jax.ShapeDtypeStruct((N, CP), jnp.float32),
    )(g1, h1d, dinv, b1r, w2p)


def _softmax_body(g2_ref, h2d_ref, dinv_ref, b2_ref, s_ref):
    s = dinv_ref[...] * (g2_ref[...] + h2d_ref[...]) + b2_ref[...]
    z = s[:, 0:SP]
    valid = lax.broadcasted_iota(jnp.int32, (RB2, SP), 1) < KC
    z = jnp.where(valid, z, -jnp.inf)
    m = jnp.max(z, axis=1, keepdims=True)
    e = jnp.where(valid, jnp.exp(z - m), 0.0)
    s_ref[...] = e / jnp.sum(e, axis=1, keepdims=True)


def _run_softmax(g2, h2d, dinv, b2r):
    grid = N // RB2
    return pl.pallas_call(
        _softmax_body,
        grid=(grid,),
        in_specs=[
            pl.BlockSpec((RB2, CP), lambda i: (i, 0)),
            pl.BlockSpec((RB2, CP), lambda i: (i, 0)),
            pl.BlockSpec((RB2, 1), lambda i: (i, 0)),
            pl.BlockSpec((1, CP), lambda i: (0, 0)),
        ],
        out_specs=pl.BlockSpec((RB2, SP), lambda i: (i, 0)),
        out_shape=jax.ShapeDtypeStruct((N, SP), jnp.float32),
    )(g2, h2d, dinv, b2r)


def _pool_body(idx_ref, val_ref, sblk_ref, sfull_ref, pb_ref, pt_ref, xb_ref,
               sw_ref, ac_ref, xt_ref, l_ref):
    pid = pl.program_id(0)
    s_b = sblk_ref[...]
    s_all = sfull_ref[...]
    sst = lax.dot_general(s_b, s_all, (((1,), (1,)), ((), ())),
                          preferred_element_type=jnp.float32)

    xb = pb_ref[:, 0:1]
    yb = pb_ref[:, 1:2]
    xa = pt_ref[0:1, :]
    ya = pt_ref[1:2, :]
    d2 = (xb - xa) ** 2 + (yb - ya) ** 2

    cols = lax.broadcasted_iota(jnp.int32, (RB, N), 1)
    v = jnp.zeros((RB, N), jnp.float32)
    for k in range(KNN):
        v = v + jnp.where(cols == idx_ref[:, k:k + 1], val_ref[:, k:k + 1], 0.0)
    bmask = (v > 0.0).astype(jnp.float32)

    lc = jnp.sum(bmask * d2 * sst) * (sw_ref[0, 0] / E)
    m_b = jnp.dot(v, s_all, preferred_element_type=jnp.float32)
    ac = lax.dot_general(s_b, m_b, (((0,), (0,)), ((), ())),
                         preferred_element_type=jnp.float32)
    xt = lax.dot_general(s_b, xb_ref[...], (((0,), (0,)), ((), ())),
                         preferred_element_type=jnp.float32)

    @pl.when(pid == 0)
    def _():
        ac_ref[...] = jnp.zeros((SP, SP), jnp.float32)
        xt_ref[...] = jnp.zeros((SP, D), jnp.float32)
        l_ref[...] = jnp.zeros((1, 1), jnp.float32)
    ac_ref[...] += ac
    xt_ref[...] += xt
    l_ref[...] += jnp.full((1, 1), 0.0) + lc


def _run_pool(idx, val, s, positions, pos_t, features, sw):
    grid = N // RB
    return pl.pallas_call(
        _pool_body,
        grid=(grid,),
        in_specs=[
            pl.BlockSpec((RB, KNN), lambda i: (i, 0)),
            pl.BlockSpec((RB, KNN), lambda i: (i, 0)),
            pl.BlockSpec((RB, SP), lambda i: (i, 0)),
            pl.BlockSpec((N, SP), lambda i: (0, 0)),
            pl.BlockSpec((RB, 2), lambda i: (i, 0)),
            pl.BlockSpec((2, N), lambda i: (0, 0)),
            pl.BlockSpec((RB, D), lambda i: (i, 0)),
            pl.BlockSpec((1, 1), lambda i: (0, 0)),
        ],
        out_specs=[
            pl.BlockSpec((SP, SP), lambda i: (0, 0)),
            pl.BlockSpec((SP, D), lambda i: (0, 0)),
            pl.BlockSpec((1, 1), lambda i: (0, 0)),
        ],
        out_shape=[
            jax.ShapeDtypeStruct((SP, SP), jnp.float32),
            jax.ShapeDtypeStruct((SP, D), jnp.float32),
            jax.ShapeDtypeStruct((1, 1), jnp.float32),
        ],
    )(idx, val, s, s, positions, pos_t, features, sw)


TPAD = 2560
TCH = 128


def _compact_body(a_ref, out_ref):
    a = a_ref[...]
    ri = lax.broadcasted_iota(jnp.int32, (SP, SP), 0)
    ci = lax.broadcasted_iota(jnp.int32, (SP, SP), 1)
    valid = (ri < KC) & (ci < KC)
    pred = (a > THRESH) & valid
    predf = pred.astype(jnp.float32)
    ult = (ri <= ci).astype(jnp.float32)
    cs = jnp.dot(predf, ult, preferred_element_type=jnp.float32)
    rowtot = cs[:, SP - 1:SP]
    lst = (ri > ci).astype(jnp.float32)
    offs = jnp.dot(lst, rowtot, preferred_element_type=jnp.float32)
    posi = jnp.where(pred, (cs + offs).astype(jnp.int32) - 1, -1)

    for c in range(TPAD // TCH):
        tv = lax.broadcasted_iota(jnp.int32, (TCH, SP, SP), 0) + c * TCH
        oh = posi[None, :, :] == tv
        r3 = lax.broadcasted_iota(jnp.int32, (TCH, SP, SP), 1)
        c3 = lax.broadcasted_iota(jnp.int32, (TCH, SP, SP), 2)
        rch = jnp.sum(jnp.sum(jnp.where(oh, r3, 0), axis=2), axis=1)
        cch = jnp.sum(jnp.sum(jnp.where(oh, c3, 0), axis=2), axis=1)
        out_ref[0, pl.ds(c * TCH, TCH)] = rch
        out_ref[1, pl.ds(c * TCH, TCH)] = cch


def _run_compact(a_coarse):
    return pl.pallas_call(
        _compact_body,
        out_shape=jax.ShapeDtypeStruct((2, TPAD), jnp.int32),
    )(a_coarse)


def kernel(features, positions, W1, b1, W2, b2,
           lambda_weight, temperature, spatial_weight):
    pos_t = positions.T
    scal = jnp.stack([lambda_weight, temperature]).reshape(1, 2)
    sw = spatial_weight.reshape(1, 1)
    b1r = b1.reshape(1, HID)
    w2p = jnp.pad(W2, ((0, 0), (0, CP - MAXC)))
    b2r = jnp.pad(b2, (0, CP - MAXC)).reshape(1, CP)

    val, idx, rows, deg, h1, lam11 = _run_topk(features, positions, pos_t, W1, scal)
    cols = idx.reshape(E)

    pos2d, pack2d, counts, sentpos = _run_pos(cols.reshape(E // 128, 128))
    counts16 = jnp.broadcast_to(counts.reshape(NW, 1), (NW, 16))
    glpack = _sc_bucket(pos2d.reshape(E), pack2d.reshape(E), sentpos)

    dinv, h1d = _run_prep(deg, h1)
    g1 = _sc_aggregate(h1d, glpack, counts16, HID)
    h2d = _run_layer2(g1, h1d, dinv, b1r, w2p)
    g2 = _sc_aggregate(h2d, glpack, counts16, CP)
    s = _run_softmax(g2, h2d, dinv, b2r)

    a_coarse, x_t, lsum = _run_pool(idx, val, s, positions, pos_t, features, sw)
    tissue = _run_compact(a_coarse)

    edge_index = jnp.stack([rows.reshape(E), cols])
    return (edge_index,
            x_t[:KC, :],
            tissue[:, :KC * KC],
            s[:, :KC],
            lsum.reshape(()),
            lam11.reshape(()))


# speedup vs baseline: 8.2967x; 1.0650x over previous
import functools

import jax
import jax.numpy as jnp
from jax import lax
from jax.experimental import pallas as pl
from jax.experimental.pallas import tpu as pltpu
from jax.experimental.pallas import tpu_sc as plsc

N = 4096
D = 512
HID = 256
MAXC = 50
KNN = 20
E = N * KNN
SPATIAL_DECAY = 50.0
THRESH = 0.1
KC = 50
CP = 128
SP = 64
RB = 256
RB2 = 512
NSC = 2
NTILE = 16
NW = NSC * NTILE
RPT = N // NW
EPW = E // NW
CH = 128
CAP = E
PB = 128
NPB = E // (PB * 128)


def _topk_body(scal_ref, fb_ref, fa_ref, pb_ref, pt_ref, w1_ref,
               val_ref, idx_ref, row_ref, deg_ref, h1_ref, lam_ref):
    pid = pl.program_id(0)
    lam = jax.nn.sigmoid(scal_ref[0, 0])
    temp = scal_ref[0, 1]

    fb = fb_ref[...]
    fa = fa_ref[...]
    sb = 1.0 / jnp.maximum(jnp.sqrt(jnp.sum(fb * fb, axis=1, keepdims=True)), 1e-12)
    sa = 1.0 / jnp.maximum(jnp.sqrt(jnp.sum(fa * fa, axis=1, keepdims=True)), 1e-12)
    sim = lax.dot_general(fb * sb, fa * sa, (((1,), (1,)), ((), ())),
                          preferred_element_type=jnp.float32)

    xb = pb_ref[:, 0:1]
    yb = pb_ref[:, 1:2]
    xa = pt_ref[0:1, :]
    ya = pt_ref[1:2, :]
    d2 = (xb - xa) ** 2 + (yb - ya) ** 2
    spatial = jnp.exp(d2 * (-1.0 / (2.0 * SPATIAL_DECAY ** 2)))

    adj = jax.nn.sigmoid((lam * sim + (1.0 - lam) * spatial) * temp)

    cols = lax.broadcasted_iota(jnp.int32, (RB, N), 1)
    a = adj
    vals = []
    idxs = []
    for _ in range(KNN):
        m = jnp.max(a, axis=1, keepdims=True)
        i = jnp.min(jnp.where(a == m, cols, N), axis=1, keepdims=True)
        vals.append(m)
        idxs.append(i)
        a = jnp.where(cols == i, -jnp.inf, a)
    tv = jnp.concatenate(vals, axis=1)
    ti = jnp.concatenate(idxs, axis=1)

    bsel = (a == -jnp.inf).astype(jnp.float32)
    dcol = jnp.sum(bsel, axis=0, keepdims=True)

    iw = ti
    sidx = []
    sval = []
    for _ in range(KNN):
        mi = jnp.min(iw, axis=1, keepdims=True)
        hit = iw == mi
        sval.append(jnp.sum(jnp.where(hit, tv, 0.0), axis=1, keepdims=True))
        sidx.append(mi)
        iw = jnp.where(hit, jnp.int32(N), iw)
    val_ref[...] = jnp.concatenate(sval, axis=1)
    idx_ref[...] = jnp.concatenate(sidx, axis=1)
    row_ref[...] = lax.broadcasted_iota(jnp.int32, (RB, KNN), 0) + pid * RB

    h1_ref[...] = jnp.dot(fb, w1_ref[...], preferred_element_type=jnp.float32)

    @pl.when(pid == 0)
    def _():
        deg_ref[...] = jnp.zeros((1, N), jnp.float32)
        lam_ref[...] = jnp.full((1, 1), 0.0) + lam
    deg_ref[...] += dcol


def _run_topk(features, positions, pos_t, W1, scal):
    grid = N // RB
    return pl.pallas_call(
        _topk_body,
        grid=(grid,),
        in_specs=[
            pl.BlockSpec((1, 2), lambda i: (0, 0)),
            pl.BlockSpec((RB, D), lambda i: (i, 0)),
            pl.BlockSpec((N, D), lambda i: (0, 0)),
            pl.BlockSpec((RB, 2), lambda i: (i, 0)),
            pl.BlockSpec((2, N), lambda i: (0, 0)),
            pl.BlockSpec((D, HID), lambda i: (0, 0)),
        ],
        out_specs=[
            pl.BlockSpec((RB, KNN), lambda i: (i, 0)),
            pl.BlockSpec((RB, KNN), lambda i: (i, 0)),
            pl.BlockSpec((RB, KNN), lambda i: (i, 0)),
            pl.BlockSpec((1, N), lambda i: (0, 0)),
            pl.BlockSpec((RB, HID), lambda i: (i, 0)),
            pl.BlockSpec((1, 1), lambda i: (0, 0)),
        ],
        out_shape=[
            jax.ShapeDtypeStruct((N, KNN), jnp.float32),
            jax.ShapeDtypeStruct((N, KNN), jnp.int32),
            jax.ShapeDtypeStruct((N, KNN), jnp.int32),
            jax.ShapeDtypeStruct((1, N), jnp.float32),
            jax.ShapeDtypeStruct((N, HID), jnp.float32),
            jax.ShapeDtypeStruct((1, 1), jnp.float32),
        ],
    )(scal, features, features, positions, pos_t, W1)


def _pos_body(dst_ref, pos_ref, pack_ref, counts_ref, sent_ref, carry):
    pid = pl.program_id(0)

    @pl.when(pid == 0)
    def _():
        carry[...] = jnp.zeros((1, NW), jnp.float32)

    dstb = dst_ref[...]
    owner = jnp.right_shift(dstb, 7)
    ri = lax.broadcasted_iota(jnp.int32, (PB, PB), 0)
    ci = lax.broadcasted_iota(jnp.int32, (PB, PB), 1)
    uinc = (ri <= ci).astype(jnp.float32)
    lstr = (ri > ci).astype(jnp.float32)

    posf = jnp.zeros((PB, 128), jnp.float32)
    tots = []
    for o in range(NW):
        eq = (owner == o).astype(jnp.float32)
        pref = jnp.dot(eq, uinc, preferred_element_type=jnp.float32)
        rowtot = pref[:, 127:128]
        rowoffs = jnp.dot(lstr, rowtot, preferred_element_type=jnp.float32)
        base = carry[0:1, o:o + 1]
        posf += eq * (pref - 1.0 + rowoffs + base + float(o) * float(CAP))
        tots.append(rowoffs[PB - 1:PB, 0:1] + rowtot[PB - 1:PB, 0:1])
    carry[...] += jnp.concatenate(tots, axis=1)
    pos_ref[...] = posf.astype(jnp.int32)

    cnt_col = lax.dot_general(_eye(NW), carry[...], (((1,), (1,)), ((), ())),
                              preferred_element_type=jnp.float32)
    sent_ref[...] = (cnt_col.astype(jnp.int32)
                     + lax.broadcasted_iota(jnp.int32, (NW, CH), 1)
                     + lax.broadcasted_iota(jnp.int32, (NW, CH), 0) * CAP)

    eids = (lax.broadcasted_iota(jnp.int32, (PB, 128), 0) * 128
            + lax.broadcasted_iota(jnp.int32, (PB, 128), 1) + pid * (PB * 128))
    src = eids // KNN
    dl = dstb & 127
    pack_ref[...] = src * 256 + dl
    counts_ref[...] = carry[...].astype(jnp.int32)


def _run_pos(dst2d):
    return pl.pallas_call(
        _pos_body,
        grid=(NPB,),
        in_specs=[pl.BlockSpec((PB, 128), lambda i: (i, 0))],
        out_specs=[
            pl.BlockSpec((PB, 128), lambda i: (i, 0)),
            pl.BlockSpec((PB, 128), lambda i: (i, 0)),
            pl.BlockSpec((1, NW), lambda i: (0, 0)),
            pl.BlockSpec((NW, CH), lambda i: (0, 0)),
        ],
        out_shape=[
            jax.ShapeDtypeStruct((E // 128, 128), jnp.int32),
            jax.ShapeDtypeStruct((E // 128, 128), jnp.int32),
            jax.ShapeDtypeStruct((1, NW), jnp.int32),
            jax.ShapeDtypeStruct((NW, CH), jnp.int32),
        ],
        scratch_shapes=[pltpu.VMEM((1, NW), jnp.float32)],
    )(dst2d)


NCHW = EPW // CH


def _sc_bucket(pos2d, pack2d, sentpos):
    mesh = plsc.VectorSubcoreMesh(core_axis_name="c", subcore_axis_name="s")

    @functools.partial(
        pl.kernel,
        out_type=jax.ShapeDtypeStruct((NW * CAP + CH,), jnp.int32),
        mesh=mesh,
        scratch_types=[
            pltpu.VMEM((NCHW, CH), jnp.int32),
            pltpu.VMEM((NCHW, CH), jnp.int32),
            pltpu.VMEM((CH,), jnp.int32),
            pltpu.VMEM((CH,), jnp.int32),
            pltpu.SemaphoreType.DMA,
        ],
    )
    def k(pos_hbm, pack_hbm, sent_hbm, gl_hbm, posb, packb, sentv, sposv, sem):
        cid = lax.axis_index("c")
        sid = lax.axis_index("s")
        w = cid * NTILE + sid

        pltpu.sync_copy(pos_hbm.at[w], posb)
        pltpu.sync_copy(pack_hbm.at[w], packb)

        def fire(ci, _):
            pltpu.async_copy(packb.at[ci], gl_hbm.at[posb.at[ci]], sem)
            return 0
        lax.fori_loop(0, NCHW, fire, 0)

        def fill(v, _):
            sentv[pl.ds(v * 16, 16)] = jnp.full((16,), RPT, jnp.int32)
            return 0
        lax.fori_loop(0, CH // 16, fill, 0)
        pltpu.sync_copy(sent_hbm.at[w], sposv)
        pltpu.async_copy(sentv, gl_hbm.at[sposv], sem)

        def drain(ci, _):
            pltpu.make_async_copy(packb.at[0], gl_hbm.at[posb.at[0]], sem).wait()
            return 0
        lax.fori_loop(0, NCHW + 1, drain, 0)

    return k(pos2d, pack2d, sentpos)


def _sc_aggregate(table, glpack, counts16, dp):
    mesh = plsc.VectorSubcoreMesh(core_axis_name="c", subcore_axis_name="s")

    @functools.partial(
        pl.kernel,
        out_type=jax.ShapeDtypeStruct((N, dp), jnp.float32),
        mesh=mesh,
        scratch_types=[
            pltpu.VMEM((CH,), jnp.int32),
            pltpu.VMEM((CH,), jnp.int32),
            pltpu.VMEM((CH,), jnp.int32),
            pltpu.VMEM((CH,), jnp.int32),
            pltpu.VMEM((CH,), jnp.int32),
            pltpu.VMEM((CH,), jnp.int32),
            pltpu.VMEM((16,), jnp.int32),
            pltpu.VMEM((CH, dp), jnp.float32),
            pltpu.VMEM((CH, dp), jnp.float32),
            pltpu.VMEM((RPT + 16, dp), jnp.float32),
            pltpu.SemaphoreType.DMA,
            pltpu.SemaphoreType.DMA,
        ],
    )
    def k(table_hbm, gl_hbm, cnt_hbm, out_hbm,
          packv0, srcslab0, dlslab0, packv1, srcslab1, dlslab1,
          cntv, rows0, rows1, acc, sem0, sem1):
        cid = lax.axis_index("c")
        sid = lax.axis_index("s")
        w = cid * NTILE + sid
        bufs = ((packv0, srcslab0, dlslab0, rows0, sem0),
                (packv1, srcslab1, dlslab1, rows1, sem1))

        def z1(i, _):
            def z2(j, _):
                acc[i, pl.ds(j * 16, 16)] = jnp.zeros((16,), jnp.float32)
                return 0
            return lax.fori_loop(0, dp // 16, z2, 0)
        lax.fori_loop(0, RPT + 16, z1, 0)

        pltpu.sync_copy(cnt_hbm.at[w], cntv)
        cnt = cntv[...][0]
        nslab = (cnt + CH - 1) // CH
        sh8 = jnp.full((16,), 8, jnp.int32)
        m255 = jnp.full((16,), 255, jnp.int32)

        def load_issue(sl, b):
            packv, srcslab, dlslab, rows, sem = bufs[b]
            pltpu.sync_copy(gl_hbm.at[pl.ds(w * CAP + sl * CH, CH)], packv)

            def unp(v, _):
                pk = packv[pl.ds(v * 16, 16)]
                srcslab[pl.ds(v * 16, 16)] = lax.shift_right_logical(pk, sh8)
                dlslab[pl.ds(v * 16, 16)] = pk & m255
                return 0
            lax.fori_loop(0, CH // 16, unp, 0)
            pltpu.async_copy(table_hbm.at[srcslab], rows, sem)

        def wait_rmw(b):
            packv, srcslab, dlslab, rows, sem = bufs[b]
            pltpu.make_async_copy(table_hbm.at[srcslab], rows, sem).wait()

            def grp(g, _):
                dv = dlslab[pl.ds(g * 16, 16)]
                for l in range(16):
                    dr = dv[l]
                    for j in range(dp // 16):
                        plsc.addupdate(acc.at[dr, pl.ds(j * 16, 16)],
                                       rows[g * 16 + l, pl.ds(j * 16, 16)])
                return 0
            lax.fori_loop(0, CH // 16, grp, 0)

        @pl.when(nslab > 0)
        def _():
            load_issue(0, 0)

        def pair(p, _):
            s1 = 2 * p + 1
            s2 = 2 * p + 2

            @pl.when(s1 < nslab)
            def _():
                load_issue(s1, 1)
            wait_rmw(0)

            @pl.when(s1 < nslab)
            def _():
                @pl.when(s2 < nslab)
                def _():
                    load_issue(s2, 0)
                wait_rmw(1)
            return 0
        lax.fori_loop(0, (nslab + 1) // 2, pair, 0)

        pltpu.sync_copy(acc.at[pl.ds(0, RPT)], out_hbm.at[pl.ds(w * RPT, RPT)])

    return k(table, glpack, counts16)


def _eye(n):
    return (lax.broadcasted_iota(jnp.int32, (n, n), 0)
            == lax.broadcasted_iota(jnp.int32, (n, n), 1)).astype(jnp.float32)


def _prep_body(deg_ref, h1_ref, dinv_ref, h1d_ref):
    degr = deg_ref[0:1, :] + 1.0
    dinvr = lax.rsqrt(jnp.maximum(degr, 1.0))
    dinv_c = lax.dot_general(_eye(RB2), dinvr, (((1,), (1,)), ((), ())),
                             preferred_element_type=jnp.float32)
    dinv_ref[...] = dinv_c
    h1d_ref[...] = h1_ref[...] * dinv_c


def _run_prep(deg, h1):
    grid = N // RB2
    return pl.pallas_call(
        _prep_body,
        grid=(grid,),
        in_specs=[
            pl.BlockSpec((1, RB2), lambda i: (0, i)),
            pl.BlockSpec((RB2, HID), lambda i: (i, 0)),
        ],
        out_specs=[
            pl.BlockSpec((RB2, 1), lambda i: (i, 0)),
            pl.BlockSpec((RB2, HID), lambda i: (i, 0)),
        ],
        out_shape=[
            jax.ShapeDtypeStruct((N, 1), jnp.float32),
            jax.ShapeDtypeStruct((N, HID), jnp.float32),
        ],
    )(deg, h1)


def _layer2_body(g1_ref, h1d_ref, dinv_ref, b1_ref, w2_ref, h2d_ref):
    dinv = dinv_ref[...]
    g = g1_ref[...] + h1d_ref[...]
    h = jnp.maximum(dinv * g + b1_ref[...], 0.0)
    h2d_ref[...] = dinv * jnp.dot(h, w2_ref[...], preferred_element_type=jnp.float32)


def _run_layer2(g1, h1d, dinv, b1r, w2p):
    grid = N // RB2
    return pl.pallas_call(
        _layer2_body,
        grid=(grid,),
        in_specs=[
            pl.BlockSpec((RB2, HID), lambda i: (i, 0)),
            pl.BlockSpec((RB2, HID), lambda i: (i, 0)),
            pl.BlockSpec((RB2, 1), lambda i: (i, 0)),
            pl.BlockSpec((1, HID), lambda i: (0, 0)),
            pl.BlockSpec((HID, CP), lambda i: (0, 0)),
        ],
        out_specs=pl.BlockSpec((RB2, CP), lambda i: (i, 0)),
        out_shape=jax.ShapeDtypeStruct((N, CP), jnp.float32),
    )(g1, h1d, dinv, b1r, w2p)


def _softmax_body(g2_ref, h2d_ref, dinv_ref, b2_ref, s_ref):
    s = dinv_ref[...] * (g2_ref[...] + h2d_ref[...]) + b2_ref[...]
    z = s[:, 0:SP]
    valid = lax.broadcasted_iota(jnp.int32, (RB2, SP), 1) < KC
    z = jnp.where(valid, z, -jnp.inf)
    m = jnp.max(z, axis=1, keepdims=True)
    e = jnp.where(valid, jnp.exp(z - m), 0.0)
    s_ref[...] = e / jnp.sum(e, axis=1, keepdims=True)


def _run_softmax(g2, h2d, dinv, b2r):
    grid = N // RB2
    return pl.pallas_call(
        _softmax_body,
        grid=(grid,),
        in_specs=[
            pl.BlockSpec((RB2, CP), lambda i: (i, 0)),
            pl.BlockSpec((RB2, CP), lambda i: (i, 0)),
            pl.BlockSpec((RB2, 1), lambda i: (i, 0)),
            pl.BlockSpec((1, CP), lambda i: (0, 0)),
        ],
        out_specs=pl.BlockSpec((RB2, SP), lambda i: (i, 0)),
        out_shape=jax.ShapeDtypeStruct((N, SP), jnp.float32),
    )(g2, h2d, dinv, b2r)


def _pool_body(idx_ref, val_ref, sblk_ref, sfull_ref, pb_ref, pt_ref, xb_ref,
               sw_ref, ac_ref, xt_ref, l_ref):
    pid = pl.program_id(0)
    s_b = sblk_ref[...]
    s_all = sfull_ref[...]
    sst = lax.dot_general(s_b, s_all, (((1,), (1,)), ((), ())),
                          preferred_element_type=jnp.float32)

    xb = pb_ref[:, 0:1]
    yb = pb_ref[:, 1:2]
    xa = pt_ref[0:1, :]
    ya = pt_ref[1:2, :]
    d2 = (xb - xa) ** 2 + (yb - ya) ** 2

    cols = lax.broadcasted_iota(jnp.int32, (RB, N), 1)
    v = jnp.zeros((RB, N), jnp.float32)
    for k in range(KNN):
        v = v + jnp.where(cols == idx_ref[:, k:k + 1], val_ref[:, k:k + 1], 0.0)
    bmask = (v > 0.0).astype(jnp.float32)

    lc = jnp.sum(bmask * d2 * sst) * (sw_ref[0, 0] / E)
    m_b = jnp.dot(v, s_all, preferred_element_type=jnp.float32)
    ac = lax.dot_general(s_b, m_b, (((0,), (0,)), ((), ())),
                         preferred_element_type=jnp.float32)
    xt = lax.dot_general(s_b, xb_ref[...], (((0,), (0,)), ((), ())),
                         preferred_element_type=jnp.float32)

    @pl.when(pid == 0)
    def _():
        ac_ref[...] = jnp.zeros((SP, SP), jnp.float32)
        xt_ref[...] = jnp.zeros((SP, D), jnp.float32)
        l_ref[...] = jnp.zeros((1, 1), jnp.float32)
    ac_ref[...] += ac
    xt_ref[...] += xt
    l_ref[...] += jnp.full((1, 1), 0.0) + lc


def _run_pool(idx, val, s, positions, pos_t, features, sw):
    grid = N // RB
    return pl.pallas_call(
        _pool_body,
        grid=(grid,),
        in_specs=[
            pl.BlockSpec((RB, KNN), lambda i: (i, 0)),
            pl.BlockSpec((RB, KNN), lambda i: (i, 0)),
            pl.BlockSpec((RB, SP), lambda i: (i, 0)),
            pl.BlockSpec((N, SP), lambda i: (0, 0)),
            pl.BlockSpec((RB, 2), lambda i: (i, 0)),
            pl.BlockSpec((2, N), lambda i: (0, 0)),
            pl.BlockSpec((RB, D), lambda i: (i, 0)),
            pl.BlockSpec((1, 1), lambda i: (0, 0)),
        ],
        out_specs=[
            pl.BlockSpec((SP, SP), lambda i: (0, 0)),
            pl.BlockSpec((SP, D), lambda i: (0, 0)),
            pl.BlockSpec((1, 1), lambda i: (0, 0)),
        ],
        out_shape=[
            jax.ShapeDtypeStruct((SP, SP), jnp.float32),
            jax.ShapeDtypeStruct((SP, D), jnp.float32),
            jax.ShapeDtypeStruct((1, 1), jnp.float32),
        ],
    )(idx, val, s, s, positions, pos_t, features, sw)


TPAD = 2560
TCH = 128


def _compact_body(a_ref, out_ref):
    a = a_ref[...]
    ri = lax.broadcasted_iota(jnp.int32, (SP, SP), 0)
    ci = lax.broadcasted_iota(jnp.int32, (SP, SP), 1)
    valid = (ri < KC) & (ci < KC)
    pred = (a > THRESH) & valid
    predf = pred.astype(jnp.float32)
    ult = (ri <= ci).astype(jnp.float32)
    cs = jnp.dot(predf, ult, preferred_element_type=jnp.float32)
    rowtot = cs[:, SP - 1:SP]
    lst = (ri > ci).astype(jnp.float32)
    offs = jnp.dot(lst, rowtot, preferred_element_type=jnp.float32)
    posi = jnp.where(pred, (cs + offs).astype(jnp.int32) - 1, -1)

    for c in range(TPAD // TCH):
        tv = lax.broadcasted_iota(jnp.int32, (TCH, SP, SP), 0) + c * TCH
        oh = posi[None, :, :] == tv
        r3 = lax.broadcasted_iota(jnp.int32, (TCH, SP, SP), 1)
        c3 = lax.broadcasted_iota(jnp.int32, (TCH, SP, SP), 2)
        rch = jnp.sum(jnp.sum(jnp.where(oh, r3, 0), axis=2), axis=1)
        cch = jnp.sum(jnp.sum(jnp.where(oh, c3, 0), axis=2), axis=1)
        out_ref[0, pl.ds(c * TCH, TCH)] = rch
        out_ref[1, pl.ds(c * TCH, TCH)] = cch


def _run_compact(a_coarse):
    return pl.pallas_call(
        _compact_body,
        out_shape=jax.ShapeDtypeStruct((2, TPAD), jnp.int32),
    )(a_coarse)


def kernel(features, positions, W1, b1, W2, b2,
           lambda_weight, temperature, spatial_weight):
    pos_t = positions.T
    scal = jnp.stack([lambda_weight, temperature]).reshape(1, 2)
    sw = spatial_weight.reshape(1, 1)
    b1r = b1.reshape(1, HID)
    w2p = jnp.pad(W2, ((0, 0), (0, CP - MAXC)))
    b2r = jnp.pad(b2, (0, CP - MAXC)).reshape(1, CP)

    val, idx, rows, deg, h1, lam11 = _run_topk(features, positions, pos_t, W1, scal)
    cols = idx.reshape(E)

    pos2d, pack2d, counts, sentpos = _run_pos(cols.reshape(E // 128, 128))
    counts16 = jnp.broadcast_to(counts.reshape(NW, 1), (NW, 16))
    glpack = _sc_bucket(pos2d.reshape(NW, NCHW, CH),
                        pack2d.reshape(NW, NCHW, CH), sentpos)

    dinv, h1d = _run_prep(deg, h1)
    g1 = _sc_aggregate(h1d, glpack, counts16, HID)
    h2d = _run_layer2(g1, h1d, dinv, b1r, w2p)
    g2 = _sc_aggregate(h2d, glpack, counts16, CP)
    s = _run_softmax(g2, h2d, dinv, b2r)

    a_coarse, x_t, lsum = _run_pool(idx, val, s, positions, pos_t, features, sw)
    tissue = _run_compact(a_coarse)

    edge_index = jnp.stack([rows.reshape(E), cols])
    return (edge_index,
            x_t[:KC, :],
            tissue[:, :KC * KC],
            s[:, :KC],
            lsum.reshape(()),
            lam11.reshape(()))
